# Initial kernel scaffold; baseline (speedup 1.0000x reference)
import jax
import jax.numpy as jnp
from jax.experimental import pallas as pl


def kernel(encodings, codebook):
    raise NotImplementedError("write your pallas kernel here")


# speedup vs baseline: 2.4602x; 2.4602x over previous
import jax
import jax.numpy as jnp
from jax.experimental import pallas as pl
from jax.experimental.pallas import tpu as pltpu

_D = 32
_K = 8192
_TN = 256


def _vq_tile_kernel(x_ref, cb_ref, out_ref):
    x = x_ref[...]
    cb = cb_ref[...]
    a = jnp.sum(x * x, axis=1, keepdims=True)
    c = jnp.sum(cb * cb, axis=0, keepdims=True)
    b = jnp.dot(x, cb, preferred_element_type=jnp.float32)
    d = (a - 2.0 * b) + c
    mv = jnp.min(d, axis=1, keepdims=True)
    lane = jax.lax.broadcasted_iota(jnp.int32, d.shape, 1)
    idx = jnp.min(jnp.where(d == mv, lane, _K), axis=1)
    one_hot = (lane == idx[:, None]).astype(jnp.float32)
    out_ref[...] = jax.lax.dot_general(
        one_hot, cb, (((1,), (1,)), ((), ())),
        preferred_element_type=jnp.float32)


def _vq_flat(x, codebook):
    n = x.shape[0]
    grid = (n // _TN,)
    return pl.pallas_call(
        _vq_tile_kernel,
        grid=grid,
        in_specs=[
            pl.BlockSpec((_TN, _D), lambda i: (i, 0)),
            pl.BlockSpec((_D, _K), lambda i: (0, 0)),
        ],
        out_specs=pl.BlockSpec((_TN, _D), lambda i: (i, 0)),
        out_shape=jax.ShapeDtypeStruct((n, _D), jnp.float32),
        compiler_params=pltpu.CompilerParams(
            dimension_semantics=("arbitrary",),
        ),
    )(x, codebook)


def kernel(encodings, codebook):
    b_, d_, h_, w_ = encodings.shape
    x = jnp.transpose(encodings, (0, 2, 3, 1)).reshape(-1, d_)
    out = _vq_flat(x, codebook)
    return jnp.transpose(out.reshape(b_, h_, w_, d_), (0, 3, 1, 2))


# speedup vs baseline: 3.7449x; 1.5222x over previous
import functools

import jax
import jax.numpy as jnp
from jax import lax
from jax.experimental import pallas as pl
from jax.experimental.pallas import tpu as pltpu
from jax.experimental.pallas import tpu_sc as plsc

_D = 32
_K = 8192
_TN = 512


def _argmin_tile_kernel(x_ref, cb2_ref, idx_ref):
    x = x_ref[...]
    cb2 = cb2_ref[...]
    a = jnp.sum(x * x, axis=1, keepdims=True)
    c = 0.25 * jnp.sum(cb2 * cb2, axis=0, keepdims=True)
    b = jnp.dot(x, cb2, preferred_element_type=jnp.float32)
    d = (a + b) + c
    mv = jnp.min(d, axis=1, keepdims=True)
    lane = lax.broadcasted_iota(jnp.int32, d.shape, 1)
    idx = jnp.min(jnp.where(d == mv, lane, _K), axis=1)
    idx_ref[0, 0, :] = idx


def _argmin_flat(x, cb2):
    n = x.shape[0]
    grid = (n // _TN,)
    idx3 = pl.pallas_call(
        _argmin_tile_kernel,
        grid=grid,
        in_specs=[
            pl.BlockSpec((_TN, _D), lambda i: (i, 0)),
            pl.BlockSpec((_D, _K), lambda i: (0, 0)),
        ],
        out_specs=pl.BlockSpec((1, 1, _TN), lambda i: (i, 0, 0)),
        out_shape=jax.ShapeDtypeStruct((n // _TN, 1, _TN), jnp.int32),
        compiler_params=pltpu.CompilerParams(
            dimension_semantics=("arbitrary",),
        ),
    )(x, cb2)
    return idx3.reshape(n)


def _gather_rows(table128, idx):
    n = idx.shape[0]
    info = plsc.get_sparse_core_info()
    nw = info.num_cores * info.num_subcores
    b_per_w = n // nw
    mesh = plsc.VectorSubcoreMesh(core_axis_name="c", subcore_axis_name="s")

    @functools.partial(
        pl.kernel,
        out_type=jax.ShapeDtypeStruct((n, 128), jnp.float32),
        mesh=mesh,
        scratch_types=[
            pltpu.VMEM((b_per_w,), jnp.int32),
            pltpu.VMEM((b_per_w, 128), jnp.float32),
            pltpu.SemaphoreType.DMA,
        ],
    )
    def gather_kernel(table_hbm, idx_hbm, out_hbm, idx_v, rows_v, sem):
        wid = lax.axis_index("s") * info.num_cores + lax.axis_index("c")
        base = wid * b_per_w
        pltpu.sync_copy(idx_hbm.at[pl.ds(base, b_per_w)], idx_v)
        pltpu.async_copy(table_hbm.at[idx_v], rows_v, sem).wait()
        pltpu.sync_copy(rows_v, out_hbm.at[pl.ds(base, b_per_w)])

    return gather_kernel(table128, idx)[:, :_D]


def kernel(encodings, codebook):
    b_, d_, h_, w_ = encodings.shape
    x = jnp.transpose(encodings, (0, 2, 3, 1)).reshape(-1, d_)
    cb2 = -2.0 * codebook
    table = codebook.T.astype(jnp.bfloat16).astype(jnp.float32)
    table128 = jnp.pad(table, ((0, 0), (0, 128 - _D)))
    idx = _argmin_flat(x, cb2)
    out = _gather_rows(table128, idx)
    return jnp.transpose(out.reshape(b_, h_, w_, d_), (0, 3, 1, 2))


# speedup vs baseline: 4.1481x; 1.1077x over previous
import functools

import jax
import jax.numpy as jnp
from jax import lax
from jax.experimental import pallas as pl
from jax.experimental.pallas import tpu as pltpu
from jax.experimental.pallas import tpu_sc as plsc

_D = 32
_K = 8192
_TN = 512


_RG = 8


def _argmin_tile_kernel(x_ref, cb2_ref, lane_ref, idx_ref):
    x = x_ref[...]
    cb2 = cb2_ref[...]
    a = jnp.sum(x * x, axis=1, keepdims=True)
    c = 0.25 * jnp.sum(cb2 * cb2, axis=0, keepdims=True)
    b = jnp.dot(x, cb2, preferred_element_type=jnp.float32)
    d = (a + b) + c
    mv = jnp.min(d, axis=1, keepdims=True)
    lane = lane_ref[...]
    idxf = jnp.min(jnp.where(d == mv, lane, float(_K)), axis=1)
    idx_ref[0, 0, :] = idxf.astype(jnp.int32)


def _argmin_flat(x, cb2, lanes):
    n = x.shape[0]
    grid = (n // _TN,)
    idx3 = pl.pallas_call(
        _argmin_tile_kernel,
        grid=grid,
        in_specs=[
            pl.BlockSpec((_TN, _D), lambda i: (i, 0)),
            pl.BlockSpec((_D, _K), lambda i: (0, 0)),
            pl.BlockSpec((1, _K), lambda i: (0, 0)),
        ],
        out_specs=pl.BlockSpec((1, 1, _TN), lambda i: (i, 0, 0)),
        out_shape=jax.ShapeDtypeStruct((n // _TN, 1, _TN), jnp.int32),
        compiler_params=pltpu.CompilerParams(
            dimension_semantics=("arbitrary",),
        ),
    )(x, cb2, lanes)
    return idx3.reshape(n)


def _gather_rows(table128, idx):
    n = idx.shape[0]
    info = plsc.get_sparse_core_info()
    nw = info.num_cores * info.num_subcores
    b_per_w = n // nw
    mesh = plsc.VectorSubcoreMesh(core_axis_name="c", subcore_axis_name="s")

    @functools.partial(
        pl.kernel,
        out_type=jax.ShapeDtypeStruct((n, 128), jnp.float32),
        mesh=mesh,
        scratch_types=[
            pltpu.VMEM((b_per_w,), jnp.int32),
            pltpu.VMEM((b_per_w, 128), jnp.float32),
            pltpu.SemaphoreType.DMA,
        ],
    )
    def gather_kernel(table_hbm, idx_hbm, out_hbm, idx_v, rows_v, sem):
        wid = lax.axis_index("s") * info.num_cores + lax.axis_index("c")
        base = wid * b_per_w
        pltpu.sync_copy(idx_hbm.at[pl.ds(base, b_per_w)], idx_v)
        pltpu.async_copy(table_hbm.at[idx_v], rows_v, sem).wait()
        pltpu.sync_copy(rows_v, out_hbm.at[pl.ds(base, b_per_w)])

    return gather_kernel(table128, idx)[:, :_D]


def kernel(encodings, codebook):
    b_, d_, h_, w_ = encodings.shape
    x = jnp.transpose(encodings, (0, 2, 3, 1)).reshape(-1, d_)
    cb2 = -2.0 * codebook
    table = codebook.T.astype(jnp.bfloat16).astype(jnp.float32)
    table128 = jnp.pad(table, ((0, 0), (0, 128 - _D)))
    lanes = jnp.arange(_K, dtype=jnp.float32).reshape(1, _K)
    idx = _argmin_flat(x, cb2, lanes)
    out = _gather_rows(table128, idx)
    return jnp.transpose(out.reshape(b_, h_, w_, d_), (0, 3, 1, 2))


# speedup vs baseline: 4.2032x; 1.0133x over previous
import functools

import jax
import jax.numpy as jnp
from jax import lax
from jax.experimental import pallas as pl
from jax.experimental.pallas import tpu as pltpu
from jax.experimental.pallas import tpu_sc as plsc

_D = 32
_K = 8192
_TN = 512


_RG = 8


def _argmin_tile_kernel(x_ref, cb2_ref, lane_ref, idx_ref):
    x = jnp.transpose(x_ref[0])
    cb2 = cb2_ref[...]
    a = jnp.sum(x * x, axis=1, keepdims=True)
    c = 0.25 * jnp.sum(cb2 * cb2, axis=0, keepdims=True)
    b = jnp.dot(x, cb2, preferred_element_type=jnp.float32)
    d = (a + b) + c
    mv = jnp.min(d, axis=1, keepdims=True)
    lane = lane_ref[...]
    im = None
    for j in range(_K // 128):
        sl = slice(j * 128, (j + 1) * 128)
        masked = jnp.where(d[:, sl] == mv, lane + float(j * 128),
                           float(_K))
        im = masked if im is None else jnp.minimum(im, masked)
    idxf = jnp.min(im, axis=1)
    idx_ref[0, 0, :] = idxf.astype(jnp.int32)


def _argmin_flat(enc3, cb2, lanes):
    n = enc3.shape[0] * enc3.shape[2]
    hw = enc3.shape[2]
    blocks_per_b = hw // _TN
    grid = (n // _TN,)
    idx3 = pl.pallas_call(
        _argmin_tile_kernel,
        grid=grid,
        in_specs=[
            pl.BlockSpec((1, _D, _TN),
                         lambda i: (i // blocks_per_b, 0, i % blocks_per_b)),
            pl.BlockSpec((_D, _K), lambda i: (0, 0)),
            pl.BlockSpec((1, 128), lambda i: (0, 0)),
        ],
        out_specs=pl.BlockSpec((1, 1, _TN), lambda i: (i, 0, 0)),
        out_shape=jax.ShapeDtypeStruct((n // _TN, 1, _TN), jnp.int32),
        compiler_params=pltpu.CompilerParams(
            dimension_semantics=("arbitrary",),
        ),
    )(enc3, cb2, lanes)
    return idx3.reshape(n)


def _gather_rows(table128, idx):
    n = idx.shape[0]
    info = plsc.get_sparse_core_info()
    nw = info.num_cores * info.num_subcores
    b_per_w = n // nw
    mesh = plsc.VectorSubcoreMesh(core_axis_name="c", subcore_axis_name="s")

    @functools.partial(
        pl.kernel,
        out_type=jax.ShapeDtypeStruct((n, 128), jnp.float32),
        mesh=mesh,
        scratch_types=[
            pltpu.VMEM((b_per_w,), jnp.int32),
            pltpu.VMEM((b_per_w, 128), jnp.float32),
            pltpu.SemaphoreType.DMA,
        ],
    )
    def gather_kernel(table_hbm, idx_hbm, out_hbm, idx_v, rows_v, sem):
        wid = lax.axis_index("s") * info.num_cores + lax.axis_index("c")
        base = wid * b_per_w
        pltpu.sync_copy(idx_hbm.at[pl.ds(base, b_per_w)], idx_v)
        pltpu.async_copy(table_hbm.at[idx_v], rows_v, sem).wait()
        pltpu.sync_copy(rows_v, out_hbm.at[pl.ds(base, b_per_w)])

    return gather_kernel(table128, idx)[:, :_D]


def kernel(encodings, codebook):
    b_, d_, h_, w_ = encodings.shape
    enc3 = encodings.reshape(b_, d_, h_ * w_)
    cb2 = -2.0 * codebook
    table = codebook.T.astype(jnp.bfloat16).astype(jnp.float32)
    table128 = jnp.pad(table, ((0, 0), (0, 128 - _D)))
    lanes = jnp.arange(128, dtype=jnp.float32).reshape(1, 128)
    idx = _argmin_flat(enc3, cb2, lanes)
    out = _gather_rows(table128, idx)
    return jnp.transpose(out.reshape(b_, h_, w_, d_), (0, 3, 1, 2))


# speedup vs baseline: 4.2782x; 1.0178x over previous
import functools

import jax
import jax.numpy as jnp
from jax import lax
from jax.experimental import pallas as pl
from jax.experimental.pallas import tpu as pltpu
from jax.experimental.pallas import tpu_sc as plsc

_D = 32
_K = 8192
_TN = 1024


_RG = 8


def _argmin_tile_kernel(x_ref, cb2_ref, lane_ref, idx_ref):
    x = jnp.transpose(x_ref[0])
    cb2 = cb2_ref[...]
    a = jnp.sum(x * x, axis=1, keepdims=True)
    c = 0.25 * jnp.sum(cb2 * cb2, axis=0, keepdims=True)
    b = jnp.dot(x, cb2, preferred_element_type=jnp.float32)
    d = (a + b) + c
    mv = jnp.min(d, axis=1, keepdims=True)
    lane = lane_ref[...]
    im = None
    for j in range(_K // 128):
        sl = slice(j * 128, (j + 1) * 128)
        masked = jnp.where(d[:, sl] == mv, lane + float(j * 128),
                           float(_K))
        im = masked if im is None else jnp.minimum(im, masked)
    idxf = jnp.min(im, axis=1)
    idx_ref[0, 0, :] = idxf.astype(jnp.int32)


def _argmin_flat(enc3, cb2, lanes):
    n = enc3.shape[0] * enc3.shape[2]
    hw = enc3.shape[2]
    blocks_per_b = hw // _TN
    grid = (n // _TN,)
    idx3 = pl.pallas_call(
        _argmin_tile_kernel,
        grid=grid,
        in_specs=[
            pl.BlockSpec((1, _D, _TN),
                         lambda i: (i // blocks_per_b, 0, i % blocks_per_b)),
            pl.BlockSpec((_D, _K), lambda i: (0, 0)),
            pl.BlockSpec((1, 128), lambda i: (0, 0)),
        ],
        out_specs=pl.BlockSpec((1, 1, _TN), lambda i: (i, 0, 0)),
        out_shape=jax.ShapeDtypeStruct((n // _TN, 1, _TN), jnp.int32),
        compiler_params=pltpu.CompilerParams(
            dimension_semantics=("arbitrary",),
        ),
    )(enc3, cb2, lanes)
    return idx3.reshape(n)


def _gather_rows(table128, idx):
    n = idx.shape[0]
    info = plsc.get_sparse_core_info()
    nw = info.num_cores * info.num_subcores
    b_per_w = n // nw
    mesh = plsc.VectorSubcoreMesh(core_axis_name="c", subcore_axis_name="s")

    @functools.partial(
        pl.kernel,
        out_type=jax.ShapeDtypeStruct((n, 128), jnp.float32),
        mesh=mesh,
        scratch_types=[
            pltpu.VMEM((b_per_w,), jnp.int32),
            pltpu.VMEM((b_per_w, 128), jnp.float32),
            pltpu.SemaphoreType.DMA,
        ],
    )
    def gather_kernel(table_hbm, idx_hbm, out_hbm, idx_v, rows_v, sem):
        wid = lax.axis_index("s") * info.num_cores + lax.axis_index("c")
        base = wid * b_per_w
        pltpu.sync_copy(idx_hbm.at[pl.ds(base, b_per_w)], idx_v)
        pltpu.async_copy(table_hbm.at[idx_v], rows_v, sem).wait()
        pltpu.sync_copy(rows_v, out_hbm.at[pl.ds(base, b_per_w)])

    return gather_kernel(table128, idx)[:, :_D]


def kernel(encodings, codebook):
    b_, d_, h_, w_ = encodings.shape
    enc3 = encodings.reshape(b_, d_, h_ * w_)
    cb2 = -2.0 * codebook
    table = codebook.T.astype(jnp.bfloat16).astype(jnp.float32)
    table128 = jnp.pad(table, ((0, 0), (0, 128 - _D)))
    lanes = jnp.arange(128, dtype=jnp.float32).reshape(1, 128)
    idx = _argmin_flat(enc3, cb2, lanes)
    out = _gather_rows(table128, idx)
    return jnp.transpose(out.reshape(b_, h_, w_, d_), (0, 3, 1, 2))


# speedup vs baseline: 4.2883x; 1.0024x over previous
import functools

import jax
import jax.numpy as jnp
from jax import lax
from jax.experimental import pallas as pl
from jax.experimental.pallas import tpu as pltpu
from jax.experimental.pallas import tpu_sc as plsc

_D = 32
_K = 8192
_TN = 1024


_RG = 8


def _argmin_tile_kernel(x_ref, cb2_ref, lane_ref, idx_ref):
    x = jnp.transpose(x_ref[0])
    cb2 = cb2_ref[...]
    a = jnp.sum(x * x, axis=1, keepdims=True)
    c = 0.25 * jnp.sum(cb2 * cb2, axis=0, keepdims=True)
    b = jnp.dot(x, cb2, preferred_element_type=jnp.float32)
    d = (a + b) + c
    mv = jnp.min(d, axis=1, keepdims=True)
    lane = lane_ref[...]
    im = None
    for j in range(_K // 128):
        sl = slice(j * 128, (j + 1) * 128)
        masked = jnp.where(d[:, sl] == mv, lane + float(j * 128),
                           float(_K))
        im = masked if im is None else jnp.minimum(im, masked)
    idxf = jnp.min(im, axis=1)
    idx_ref[0, 0, :] = idxf.astype(jnp.int32)


def _argmin_flat(enc3, cb2, lanes, b0, nb):
    hw = enc3.shape[2]
    n = nb * hw
    blocks_per_b = hw // _TN
    grid = (n // _TN,)
    idx3 = pl.pallas_call(
        _argmin_tile_kernel,
        grid=grid,
        in_specs=[
            pl.BlockSpec((1, _D, _TN),
                         lambda i: (b0 + i // blocks_per_b, 0,
                                    i % blocks_per_b)),
            pl.BlockSpec((_D, _K), lambda i: (0, 0)),
            pl.BlockSpec((1, 128), lambda i: (0, 0)),
        ],
        out_specs=pl.BlockSpec((1, 1, _TN), lambda i: (i, 0, 0)),
        out_shape=jax.ShapeDtypeStruct((n // _TN, 1, _TN), jnp.int32),
        compiler_params=pltpu.CompilerParams(
            dimension_semantics=("arbitrary",),
        ),
    )(enc3, cb2, lanes)
    return idx3.reshape(n)


def _gather_rows(table128, idx):
    n = idx.shape[0]
    info = plsc.get_sparse_core_info()
    nw = info.num_cores * info.num_subcores
    b_per_w = n // nw
    mesh = plsc.VectorSubcoreMesh(core_axis_name="c", subcore_axis_name="s")

    @functools.partial(
        pl.kernel,
        out_type=jax.ShapeDtypeStruct((n, 128), jnp.float32),
        mesh=mesh,
        scratch_types=[
            pltpu.VMEM((b_per_w,), jnp.int32),
            pltpu.VMEM((b_per_w, 128), jnp.float32),
            pltpu.SemaphoreType.DMA,
        ],
    )
    def gather_kernel(table_hbm, idx_hbm, out_hbm, idx_v, rows_v, sem):
        wid = lax.axis_index("s") * info.num_cores + lax.axis_index("c")
        base = wid * b_per_w
        pltpu.sync_copy(idx_hbm.at[pl.ds(base, b_per_w)], idx_v)
        pltpu.async_copy(table_hbm.at[idx_v], rows_v, sem).wait()
        pltpu.sync_copy(rows_v, out_hbm.at[pl.ds(base, b_per_w)])

    return gather_kernel(table128, idx)[:, :_D]


def kernel(encodings, codebook):
    b_, d_, h_, w_ = encodings.shape
    enc3 = encodings.reshape(b_, d_, h_ * w_)
    cb2 = -2.0 * codebook
    table = codebook.T.astype(jnp.bfloat16).astype(jnp.float32)
    table128 = jnp.pad(table, ((0, 0), (0, 128 - _D)))
    lanes = jnp.arange(128, dtype=jnp.float32).reshape(1, 128)
    halves = []
    hb = b_ // 2
    for b0 in (0, hb):
        idx = _argmin_flat(enc3, cb2, lanes, b0, hb)
        out = _gather_rows(table128, idx)
        halves.append(
            jnp.transpose(out.reshape(hb, h_, w_, d_), (0, 3, 1, 2)))
    return jnp.concatenate(halves, axis=0)


# speedup vs baseline: 4.2997x; 1.0027x over previous
import functools

import jax
import jax.numpy as jnp
from jax import lax
from jax.experimental import pallas as pl
from jax.experimental.pallas import tpu as pltpu
from jax.experimental.pallas import tpu_sc as plsc

_D = 32
_K = 8192
_TN = 1024


def _argmin_tile_kernel(x_ref, cb2_ref, lane_ref, idx_ref):
    x = jnp.transpose(x_ref[0])
    cb2 = cb2_ref[...]
    a = jnp.sum(x * x, axis=1, keepdims=True)
    c = 0.25 * jnp.sum(cb2 * cb2, axis=0, keepdims=True)
    b = jnp.dot(x, cb2, preferred_element_type=jnp.float32)
    d = (a + b) + c
    mv = jnp.min(d, axis=1, keepdims=True)
    lane = lane_ref[...]
    im = None
    for j in range(_K // 128):
        sl = slice(j * 128, (j + 1) * 128)
        masked = jnp.where(d[:, sl] == mv, lane + float(j * 128),
                           float(_K))
        im = masked if im is None else jnp.minimum(im, masked)
    idxf = jnp.min(im, axis=1)
    idx_ref[0, 0, :] = idxf.astype(jnp.int32)


def _argmin_flat(enc3, cb2, lanes, b0, nb):
    hw = enc3.shape[2]
    n = nb * hw
    blocks_per_b = hw // _TN
    grid = (n // _TN,)
    idx3 = pl.pallas_call(
        _argmin_tile_kernel,
        grid=grid,
        in_specs=[
            pl.BlockSpec((1, _D, _TN),
                         lambda i: (b0 + i // blocks_per_b, 0,
                                    i % blocks_per_b)),
            pl.BlockSpec((_D, _K), lambda i: (0, 0)),
            pl.BlockSpec((1, 128), lambda i: (0, 0)),
        ],
        out_specs=pl.BlockSpec((1, 1, _TN), lambda i: (i, 0, 0)),
        out_shape=jax.ShapeDtypeStruct((n // _TN, 1, _TN), jnp.int32),
        compiler_params=pltpu.CompilerParams(
            dimension_semantics=("arbitrary",),
        ),
    )(enc3, cb2, lanes)
    return idx3.reshape(n)


def _gather_rows(table128, idx):
    n = idx.shape[0]
    info = plsc.get_sparse_core_info()
    nw = info.num_cores * info.num_subcores
    b_per_w = n // nw
    mesh = plsc.VectorSubcoreMesh(core_axis_name="c", subcore_axis_name="s")

    @functools.partial(
        pl.kernel,
        out_type=jax.ShapeDtypeStruct((n, 128), jnp.float32),
        mesh=mesh,
        scratch_types=[
            pltpu.VMEM((b_per_w,), jnp.int32),
            pltpu.VMEM((b_per_w, 128), jnp.float32),
            pltpu.SemaphoreType.DMA,
        ],
    )
    def gather_kernel(table_hbm, idx_hbm, out_hbm, idx_v, rows_v, sem):
        wid = lax.axis_index("s") * info.num_cores + lax.axis_index("c")
        base = wid * b_per_w
        pltpu.sync_copy(idx_hbm.at[pl.ds(base, b_per_w)], idx_v)
        pltpu.async_copy(table_hbm.at[idx_v], rows_v, sem).wait()
        pltpu.sync_copy(rows_v, out_hbm.at[pl.ds(base, b_per_w)])

    return gather_kernel(table128, idx)[:, :_D]


def kernel(encodings, codebook):
    b_, d_, h_, w_ = encodings.shape
    enc3 = encodings.reshape(b_, d_, h_ * w_)
    cb2 = -2.0 * codebook
    table = codebook.T.astype(jnp.bfloat16).astype(jnp.float32)
    table128 = jnp.pad(table, ((0, 0), (0, 128 - _D)))
    lanes = jnp.arange(128, dtype=jnp.float32).reshape(1, 128)
    halves = []
    hb = b_ // 2
    for b0 in (0, hb):
        idx = _argmin_flat(enc3, cb2, lanes, b0, hb)
        out = _gather_rows(table128, idx)
        halves.append(
            jnp.transpose(out.reshape(hb, h_, w_, d_), (0, 3, 1, 2)))
    return jnp.concatenate(halves, axis=0)
